# Initial kernel scaffold; baseline (speedup 1.0000x reference)
import functools

import jax
import jax.numpy as jnp
from jax import lax
from jax.experimental import pallas as pl
from jax.experimental.pallas import tpu as pltpu
from jax.experimental.pallas import tpu_sc as plsc

N_NODES = 50000
N_ACC = 50176
PT_ROWS = N_ACC // 16
ZROWS = 392
CHUNK = 128
EPT_CHUNKS = 416
EPT = CHUNK * EPT_CHUNKS
E_PAD = EPT * 16


def _make_propagate(nb, w):
    mesh = plsc.VectorSubcoreMesh(core_axis_name="c", subcore_axis_name="s")
    out_t = [jax.ShapeDtypeStruct((N_ACC, w), jnp.float32) for _ in range(nb)]
    scratch = [
        pltpu.VMEM((CHUNK,), jnp.int32),
        pltpu.VMEM((CHUNK,), jnp.int32),
        pltpu.VMEM((CHUNK, w), jnp.float32),
        pltpu.VMEM((ZROWS, w), jnp.float32),
        pltpu.VMEM((ZROWS, w), jnp.float32),
        pltpu.VMEM_SHARED((N_ACC, w), jnp.float32),
        pltpu.SemaphoreType.DMA,
    ]

    @functools.partial(pl.kernel, mesh=mesh, out_type=out_t,
                       scratch_types=scratch)
    def kern(row_hbm, col_hbm, zeros_hbm, *rest):
        tabs = rest[:nb]
        outs = rest[nb:2 * nb]
        row_v, col_v, rows_v, zbuf, fbuf, acc, sem = rest[2 * nb:]
        cid = lax.axis_index("c")
        sid = lax.axis_index("s")
        pltpu.sync_copy(zeros_hbm, zbuf)
        for b in range(nb):
            target = b % 2 if nb > 1 else 0

            @pl.when(cid == target)
            def _block(b=b):
                for z in range(PT_ROWS // ZROWS):
                    pltpu.sync_copy(
                        zbuf, acc.at[pl.ds(sid * PT_ROWS + z * ZROWS, ZROWS)])
                plsc.subcore_barrier()

                def body(i, carry):
                    base = sid * EPT + i * CHUNK
                    pltpu.sync_copy(row_hbm.at[pl.ds(base, CHUNK)], row_v)
                    pltpu.sync_copy(col_hbm.at[pl.ds(base, CHUNK)], col_v)
                    pltpu.async_copy(tabs[b].at[row_v], rows_v, sem).wait()
                    pltpu.sync_copy(rows_v, acc.at[col_v], add=True)
                    return carry

                lax.fori_loop(0, EPT_CHUNKS, body, 0)
                plsc.subcore_barrier()
                for z in range(PT_ROWS // ZROWS):
                    off = sid * PT_ROWS + z * ZROWS
                    pltpu.sync_copy(acc.at[pl.ds(off, ZROWS)], fbuf)
                    pltpu.sync_copy(fbuf, outs[b].at[pl.ds(off, ZROWS)])
                plsc.subcore_barrier()

    return kern


_prop_w1 = _make_propagate(1, 1)
_prop_w4 = _make_propagate(1, 4)
_prop_2x32 = _make_propagate(2, 32)
_prop_4x32 = _make_propagate(4, 32)


def _run_prop(kern, row, col, zeros, tabs):
    res = kern(row, col, zeros, *tabs)
    if not isinstance(res, (list, tuple)):
        res = (res,)
    return res


R = 1000
GRID = (N_NODES // R,)


def _rows(w):
    return pl.BlockSpec((R, w), lambda i: (i, 0))


def _full(shape):
    return pl.BlockSpec(shape, lambda i: tuple(0 for _ in shape))


def _tc_call(body, in_specs, out_w):
    return pl.pallas_call(
        body,
        grid=GRID,
        in_specs=in_specs,
        out_specs=[_rows(w) for w in out_w],
        out_shape=[jax.ShapeDtypeStruct((N_NODES, w), jnp.float32)
                   for w in out_w],
    )


def _k0_body(degS, xpad, dinv_o, xp_o):
    dv = lax.rsqrt(degS[...])
    dinv_o[...] = dv
    xp_o[...] = xpad[...] * dv


def _k1_body(t4, dinv, w1, b1, g0, g1, g2, g3):
    dv = dinv[...]
    h = jnp.maximum(jnp.dot(t4[...] * dv, w1[...],
                            preferred_element_type=jnp.float32) + b1[...], 0.0)
    g = h * dv
    for k, o in enumerate((g0, g1, g2, g3)):
        o[...] = g[:, k * 32:(k + 1) * 32]


def _k2_body(t0, t1, t2, t3, dinv, w2, b2, w3, g0, g1):
    dv = dinv[...]
    t = jnp.concatenate([t0[...], t1[...], t2[...], t3[...]], axis=1) * dv
    h = jnp.maximum(jnp.dot(t, w2[...],
                            preferred_element_type=jnp.float32) + b2[...], 0.0)
    g = jnp.dot(h, w3[...], preferred_element_type=jnp.float32) * dv
    g0[...] = g[:, :32]
    g1[...] = g[:, 32:]


def _k3_body(t0, t1, dinv, b3, g0, g1):
    dv = dinv[...]
    t = jnp.concatenate([t0[...], t1[...]], axis=1) * dv
    g = jnp.maximum(t + b3[...], 0.0) * dv
    g0[...] = g[:, :32]
    g1[...] = g[:, 32:]


def _k4_body(t0, t1, dinv, w4, b4, w5, g_o):
    dv = dinv[...]
    t = jnp.concatenate([t0[...], t1[...]], axis=1) * dv
    h = jnp.maximum(jnp.dot(t, w4[...],
                            preferred_element_type=jnp.float32) + b4[...], 0.0)
    g_o[...] = jnp.dot(h, w5[...], preferred_element_type=jnp.float32) * dv


def _k5_body(t1, dinv, b5, out_o):
    out_o[...] = t1[...] * dinv[...] + b5[0, 0]


def kernel(x, edge_index, W1, b1, W2, b2, W3, b3, W4, b4, W5, b5):
    n = x.shape[0]
    f32 = jnp.float32
    loops = jnp.arange(n, dtype=jnp.int32)
    row = jnp.concatenate([edge_index[0].astype(jnp.int32), loops])
    col = jnp.concatenate([edge_index[1].astype(jnp.int32), loops])
    npad = E_PAD - row.shape[0]
    row = jnp.concatenate([row, jnp.zeros((npad,), jnp.int32)])
    col = jnp.concatenate([col, jnp.full((npad,), N_NODES, jnp.int32)])

    z1 = jnp.zeros((ZROWS, 1), f32)
    z4 = jnp.zeros((ZROWS, 4), f32)
    z32 = jnp.zeros((ZROWS, 32), f32)
    ones = jnp.ones((n, 1), f32)
    xpad = jnp.pad(x, ((0, 0), (0, 1)))
    W1p = jnp.pad(W1, ((0, 1), (0, 0)))
    b1r = b1.reshape(1, -1)
    b2r = b2.reshape(1, -1)
    b3r = b3.reshape(1, -1)
    b4r = b4.reshape(1, -1)
    b5r = b5.reshape(1, 1)

    (degS,) = _run_prop(_prop_w1, row, col, z1, [ones])
    dinv, xp = _tc_call(
        _k0_body, [_rows(1), _rows(4)], [1, 4])(degS, xpad)

    (t4,) = _run_prop(_prop_w4, row, col, z4, [xp])
    g1 = _tc_call(
        _k1_body,
        [_rows(4), _rows(1), _full((4, 128)), _full((1, 128))],
        [32, 32, 32, 32])(t4, dinv, W1p, b1r)

    t128 = _run_prop(_prop_4x32, row, col, z32, list(g1))
    g2 = _tc_call(
        _k2_body,
        [_rows(32)] * 4 + [_rows(1), _full((128, 128)), _full((1, 128)),
                           _full((128, 64))],
        [32, 32])(*t128, dinv, W2, b2r, W3)

    t64a = _run_prop(_prop_2x32, row, col, z32, list(g2))
    g3 = _tc_call(
        _k3_body,
        [_rows(32)] * 2 + [_rows(1), _full((1, 64))],
        [32, 32])(*t64a, dinv, b3r)

    t64b = _run_prop(_prop_2x32, row, col, z32, list(g3))
    (g4,) = _tc_call(
        _k4_body,
        [_rows(32)] * 2 + [_rows(1), _full((64, 64)), _full((1, 64)),
                           _full((64, 1))],
        [1])(*t64b, dinv, W4, b4r, W5)

    (t1,) = _run_prop(_prop_w1, row, col, z1, [g4])
    (out,) = _tc_call(
        _k5_body, [_rows(1), _rows(1), _full((1, 1))], [1])(t1, dinv, b5r)
    return out


# speedup vs baseline: 6.5908x; 6.5908x over previous
import functools

import jax
import jax.numpy as jnp
from jax import lax
from jax.experimental import pallas as pl
from jax.experimental.pallas import tpu as pltpu
from jax.experimental.pallas import tpu_sc as plsc

N_NODES = 50000
N_ACC = 50176
PT_ROWS = N_ACC // 16
ZROWS = 392
CHUNK = 128
EPT_CHUNKS = 416
EPT = CHUNK * EPT_CHUNKS
E_PAD = EPT * 16


def _make_propagate(nb, w):
    mesh = plsc.VectorSubcoreMesh(core_axis_name="c", subcore_axis_name="s")
    out_t = [jax.ShapeDtypeStruct((N_ACC, w), jnp.float32) for _ in range(nb)]
    scratch = [
        pltpu.VMEM((CHUNK,), jnp.int32),
        pltpu.VMEM((CHUNK,), jnp.int32),
        pltpu.VMEM((CHUNK, w), jnp.float32),
        pltpu.VMEM((ZROWS, w), jnp.float32),
        pltpu.VMEM((ZROWS, w), jnp.float32),
        pltpu.VMEM_SHARED((N_ACC, w), jnp.float32),
        pltpu.SemaphoreType.DMA,
    ]

    @functools.partial(
        pl.kernel, mesh=mesh, out_type=out_t, scratch_types=scratch,
        compiler_params=pltpu.CompilerParams(use_tc_tiling_on_sc=False))
    def kern(row_hbm, col_hbm, zeros_hbm, *rest):
        tabs = rest[:nb]
        outs = rest[nb:2 * nb]
        row_v, col_v, rows_v, zbuf, fbuf, acc, sem = rest[2 * nb:]
        cid = lax.axis_index("c")
        sid = lax.axis_index("s")
        pltpu.sync_copy(zeros_hbm, zbuf)
        for b in range(nb):
            target = b % 2 if nb > 1 else 0

            @pl.when(cid == target)
            def _block(b=b):
                for z in range(PT_ROWS // ZROWS):
                    pltpu.sync_copy(
                        zbuf, acc.at[pl.ds(sid * PT_ROWS + z * ZROWS, ZROWS)])
                plsc.subcore_barrier()

                def body(i, carry):
                    base = sid * EPT + i * CHUNK
                    pltpu.sync_copy(row_hbm.at[pl.ds(base, CHUNK)], row_v)
                    pltpu.sync_copy(col_hbm.at[pl.ds(base, CHUNK)], col_v)
                    pltpu.async_copy(tabs[b].at[row_v], rows_v, sem).wait()
                    pltpu.sync_copy(rows_v, acc.at[col_v], add=True)
                    return carry

                lax.fori_loop(0, EPT_CHUNKS, body, 0)
                plsc.subcore_barrier()
                for z in range(PT_ROWS // ZROWS):
                    off = sid * PT_ROWS + z * ZROWS
                    pltpu.sync_copy(acc.at[pl.ds(off, ZROWS)], fbuf)
                    pltpu.sync_copy(fbuf, outs[b].at[pl.ds(off, ZROWS)])
                plsc.subcore_barrier()

    return kern


_prop_w1 = _make_propagate(1, 1)
_prop_w4 = _make_propagate(1, 4)
_prop_2x32 = _make_propagate(2, 32)
_prop_4x32 = _make_propagate(4, 32)


def _run_prop(kern, row, col, zeros, tabs):
    res = kern(row, col, zeros, *tabs)
    if not isinstance(res, (list, tuple)):
        res = (res,)
    return res


R = 1000
GRID = (N_NODES // R,)


def _rows(w):
    return pl.BlockSpec((R, w), lambda i: (i, 0))


def _full(shape):
    return pl.BlockSpec(shape, lambda i: tuple(0 for _ in shape))


def _tc_call(body, in_specs, out_w):
    return pl.pallas_call(
        body,
        grid=GRID,
        in_specs=in_specs,
        out_specs=[_rows(w) for w in out_w],
        out_shape=[jax.ShapeDtypeStruct((N_NODES, w), jnp.float32)
                   for w in out_w],
    )


def _k0_body(degS, xpad, dinv_o, xp_o):
    dv = lax.rsqrt(degS[...])
    dinv_o[...] = dv
    xp_o[...] = xpad[...] * dv


def _k1_body(t4, dinv, w1, b1, g0, g1, g2, g3):
    dv = dinv[...]
    h = jnp.maximum(jnp.dot(t4[...] * dv, w1[...],
                            preferred_element_type=jnp.float32) + b1[...], 0.0)
    g = h * dv
    for k, o in enumerate((g0, g1, g2, g3)):
        o[...] = g[:, k * 32:(k + 1) * 32]


def _k2_body(t0, t1, t2, t3, dinv, w2, b2, w3, g0, g1):
    dv = dinv[...]
    t = jnp.concatenate([t0[...], t1[...], t2[...], t3[...]], axis=1) * dv
    h = jnp.maximum(jnp.dot(t, w2[...],
                            preferred_element_type=jnp.float32) + b2[...], 0.0)
    g = jnp.dot(h, w3[...], preferred_element_type=jnp.float32) * dv
    g0[...] = g[:, :32]
    g1[...] = g[:, 32:]


def _k3_body(t0, t1, dinv, b3, g0, g1):
    dv = dinv[...]
    t = jnp.concatenate([t0[...], t1[...]], axis=1) * dv
    g = jnp.maximum(t + b3[...], 0.0) * dv
    g0[...] = g[:, :32]
    g1[...] = g[:, 32:]


def _k4_body(t0, t1, dinv, w4, b4, w5, g_o):
    dv = dinv[...]
    t = jnp.concatenate([t0[...], t1[...]], axis=1) * dv
    h = jnp.maximum(jnp.dot(t, w4[...],
                            preferred_element_type=jnp.float32) + b4[...], 0.0)
    g_o[...] = jnp.dot(h, w5[...], preferred_element_type=jnp.float32) * dv


def _k5_body(t1, dinv, b5, out_o):
    out_o[...] = t1[...] * dinv[...] + b5[0, 0]


def kernel(x, edge_index, W1, b1, W2, b2, W3, b3, W4, b4, W5, b5):
    n = x.shape[0]
    f32 = jnp.float32
    loops = jnp.arange(n, dtype=jnp.int32)
    row = jnp.concatenate([edge_index[0].astype(jnp.int32), loops])
    col = jnp.concatenate([edge_index[1].astype(jnp.int32), loops])
    npad = E_PAD - row.shape[0]
    row = jnp.concatenate([row, jnp.zeros((npad,), jnp.int32)])
    col = jnp.concatenate([col, jnp.full((npad,), N_NODES, jnp.int32)])

    z1 = jnp.zeros((ZROWS, 1), f32)
    z4 = jnp.zeros((ZROWS, 4), f32)
    z32 = jnp.zeros((ZROWS, 32), f32)
    ones = jnp.ones((n, 1), f32)
    xpad = jnp.pad(x, ((0, 0), (0, 1)))
    W1p = jnp.pad(W1, ((0, 1), (0, 0)))
    b1r = b1.reshape(1, -1)
    b2r = b2.reshape(1, -1)
    b3r = b3.reshape(1, -1)
    b4r = b4.reshape(1, -1)
    b5r = b5.reshape(1, 1)

    (degS,) = _run_prop(_prop_w1, row, col, z1, [ones])
    dinv, xp = _tc_call(
        _k0_body, [_rows(1), _rows(4)], [1, 4])(degS, xpad)

    (t4,) = _run_prop(_prop_w4, row, col, z4, [xp])
    g1 = _tc_call(
        _k1_body,
        [_rows(4), _rows(1), _full((4, 128)), _full((1, 128))],
        [32, 32, 32, 32])(t4, dinv, W1p, b1r)

    t128 = _run_prop(_prop_4x32, row, col, z32, list(g1))
    g2 = _tc_call(
        _k2_body,
        [_rows(32)] * 4 + [_rows(1), _full((128, 128)), _full((1, 128)),
                           _full((128, 64))],
        [32, 32])(*t128, dinv, W2, b2r, W3)

    t64a = _run_prop(_prop_2x32, row, col, z32, list(g2))
    g3 = _tc_call(
        _k3_body,
        [_rows(32)] * 2 + [_rows(1), _full((1, 64))],
        [32, 32])(*t64a, dinv, b3r)

    t64b = _run_prop(_prop_2x32, row, col, z32, list(g3))
    (g4,) = _tc_call(
        _k4_body,
        [_rows(32)] * 2 + [_rows(1), _full((64, 64)), _full((1, 64)),
                           _full((64, 1))],
        [1])(*t64b, dinv, W4, b4r, W5)

    (t1,) = _run_prop(_prop_w1, row, col, z1, [g4])
    (out,) = _tc_call(
        _k5_body, [_rows(1), _rows(1), _full((1, 1))], [1])(t1, dinv, b5r)
    return out
